# Initial kernel scaffold; baseline (speedup 1.0000x reference)
import jax
import jax.numpy as jnp
from jax.experimental import pallas as pl


def kernel(X, edge_index, edge_weight, skip, H, C, params):
    raise NotImplementedError("write your pallas kernel here")


# speedup vs baseline: 8.2791x; 8.2791x over previous
import functools

import jax
import jax.numpy as jnp
from jax import lax
from jax.experimental import pallas as pl
from jax.experimental.pallas import tpu as pltpu
from jax.experimental.pallas import tpu_sc as plsc

NC = 2
NS = 16
LANES = 16
CHUNK = 128


def _sc_body(table, src, dst, ew, out, src_buf, dst_buf, ew_buf, rows, rows2,
             zbuf, acc, sem, *, cps, rpw, npad):
    cid = lax.axis_index("c")
    sid = lax.axis_index("s")
    zero = jnp.zeros((LANES,), jnp.float32)

    def zrow(r, _):
        for k in range(8):
            zbuf[r, pl.ds(k * LANES, LANES)] = zero
        return 0
    lax.fori_loop(0, zbuf.shape[0], zrow, 0)
    zrows = zbuf.shape[0]
    for b in range(rpw // zrows):
        r0 = pl.multiple_of(sid * rpw + b * zrows, zrows)
        pltpu.sync_copy(zbuf, acc.at[pl.ds(r0, zrows)])
    plsc.subcore_barrier()

    ebase = (cid * NS + sid) * (cps * CHUNK)

    def chunk(ch, _):
        e0 = pl.multiple_of(ebase + ch * CHUNK, CHUNK)
        pltpu.sync_copy(src.at[pl.ds(e0, CHUNK)], src_buf)
        pltpu.sync_copy(dst.at[pl.ds(e0, CHUNK)], dst_buf)
        pltpu.sync_copy(ew.at[pl.ds(e0, CHUNK)], ew_buf)
        pltpu.async_copy(table.at[src_buf], rows, sem).wait()

        dn = lax.GatherDimensionNumbers(
            offset_dims=(), collapsed_slice_dims=(0,), start_index_map=(0,))

        def group(g, _):
            g0 = pl.multiple_of(g * LANES, LANES)
            ewv = ew_buf[pl.ds(g0, LANES)]
            for j in range(LANES):
                r = g * LANES + j
                w = lax.gather(ewv, jnp.full((LANES, 1), j, jnp.int32),
                               dn, (1,),
                               mode=lax.GatherScatterMode.PROMISE_IN_BOUNDS)
                for k in range(8):
                    sl = pl.ds(k * LANES, LANES)
                    rows2[r, sl] = rows[r, sl] * w
            return 0
        lax.fori_loop(0, CHUNK // LANES, group, 0)
        pltpu.sync_copy(rows2, acc.at[dst_buf], add=True)
        return 0
    lax.fori_loop(0, cps, chunk, 0)
    plsc.subcore_barrier()

    a0 = pl.multiple_of(sid * rpw, 8)
    o0 = pl.multiple_of(cid * npad + sid * rpw, 8)
    pltpu.sync_copy(acc.at[pl.ds(a0, rpw)], out.at[pl.ds(o0, rpw)])


def _sc_aggregate(table, src2, dst2, ew2, *, cps, rpw, npad):
    mesh = plsc.VectorSubcoreMesh(core_axis_name="c", subcore_axis_name="s")
    kfn = pl.kernel(
        functools.partial(_sc_body, cps=cps, rpw=rpw, npad=npad),
        out_type=jax.ShapeDtypeStruct((NC * npad, 128), jnp.float32),
        mesh=mesh,
        scratch_types=[
            pltpu.VMEM((CHUNK,), jnp.int32),
            pltpu.VMEM((CHUNK,), jnp.int32),
            pltpu.VMEM((CHUNK,), jnp.float32),
            pltpu.VMEM((CHUNK, 128), jnp.float32),
            pltpu.VMEM((CHUNK, 128), jnp.float32),
            pltpu.VMEM((64, 128), jnp.float32),
            pltpu.VMEM_SHARED((npad, 128), jnp.float32),
            pltpu.SemaphoreType.DMA,
        ],
    )
    return kfn(table, src2, dst2, ew2)


def _ln(x, g, b):
    mu = jnp.mean(x, axis=-1, keepdims=True)
    var = jnp.mean((x - mu) ** 2, axis=-1, keepdims=True)
    return (x - mu) * jax.lax.rsqrt(var + 1e-5) * g + b


def _tc_cell_body(ax_ref, ah_ref, c_ref, wx_ref, wh_ref, b_ref,
                  gh_ref, bh_ref, gc_ref, bc_ref, hn_ref, cn_ref):
    gates = (jnp.dot(ax_ref[...], wx_ref[...], preferred_element_type=jnp.float32)
             + jnp.dot(ah_ref[...], wh_ref[...], preferred_element_type=jnp.float32)
             + b_ref[...])
    i = jax.nn.sigmoid(gates[:, 0:128])
    f = jax.nn.sigmoid(gates[:, 128:256])
    g = jnp.tanh(gates[:, 256:384])
    o = jax.nn.sigmoid(gates[:, 384:512])
    c_new = f * c_ref[...] + i * g
    h_new = o * jnp.tanh(c_new)
    hn_ref[...] = _ln(h_new, gh_ref[...], bh_ref[...])
    cn_ref[...] = _ln(c_new, gc_ref[...], bc_ref[...])


def _tc_final_body(ax_ref, ah_ref, c_ref, wx_ref, wh_ref, b_ref,
                   gh_ref, bh_ref, gc_ref, bc_ref, go_ref, bo_ref,
                   skip_ref, w1a_ref, w1b_ref, b1_ref, w2_ref, b2_ref,
                   hn_ref, cn_ref, out_ref):
    gates = (jnp.dot(ax_ref[...], wx_ref[...], preferred_element_type=jnp.float32)
             + jnp.dot(ah_ref[...], wh_ref[...], preferred_element_type=jnp.float32)
             + b_ref[...])
    i = jax.nn.sigmoid(gates[:, 0:128])
    f = jax.nn.sigmoid(gates[:, 128:256])
    g = jnp.tanh(gates[:, 256:384])
    o = jax.nn.sigmoid(gates[:, 384:512])
    c_new = f * c_ref[...] + i * g
    h_new = o * jnp.tanh(c_new)
    hn_ref[...] = _ln(h_new, gh_ref[...], bh_ref[...])
    cn_ref[...] = _ln(c_new, gc_ref[...], bc_ref[...])
    y = jax.nn.relu(_ln(h_new, go_ref[...], bo_ref[...]))
    t = (jnp.dot(y, w1a_ref[...], preferred_element_type=jnp.float32)
         + skip_ref[...] * w1b_ref[...] + b1_ref[...])
    t = jax.nn.relu(t)
    out_ref[...] = jax.nn.sigmoid(
        jnp.dot(t, w2_ref[...], preferred_element_type=jnp.float32) + b2_ref[...])


def _row_spec(bm, width):
    return pl.BlockSpec((bm, width), lambda i: (i, 0))


def _full_spec(shape):
    return pl.BlockSpec(shape, lambda i: tuple(0 for _ in shape))


def _tc_cell(ax, ah, c, wx, wh, b, gh, bh, gc, bc, *, bm):
    n = ax.shape[0]
    grid = (n // bm,)
    out_shape = [jax.ShapeDtypeStruct((n, 128), jnp.float32)] * 2
    return pl.pallas_call(
        _tc_cell_body,
        grid=grid,
        in_specs=[_row_spec(bm, 128), _row_spec(bm, 128), _row_spec(bm, 128),
                  _full_spec((128, 512)), _full_spec((128, 512)),
                  _full_spec((1, 512)),
                  _full_spec((1, 128)), _full_spec((1, 128)),
                  _full_spec((1, 128)), _full_spec((1, 128))],
        out_specs=[_row_spec(bm, 128), _row_spec(bm, 128)],
        out_shape=out_shape,
        compiler_params=pltpu.CompilerParams(
            dimension_semantics=("arbitrary",)),
    )(ax, ah, c, wx, wh, b.reshape(1, 512),
      gh.reshape(1, 128), bh.reshape(1, 128),
      gc.reshape(1, 128), bc.reshape(1, 128))


def _tc_final(ax, ah, c, wx, wh, b, gh, bh, gc, bc, go, bo,
              skip, w1, b1, w2, b2, *, bm):
    n = ax.shape[0]
    grid = (n // bm,)
    out_shape = [jax.ShapeDtypeStruct((n, 128), jnp.float32),
                 jax.ShapeDtypeStruct((n, 128), jnp.float32),
                 jax.ShapeDtypeStruct((n, 128), jnp.float32)]
    skip128 = jnp.broadcast_to(skip, (n, 128))
    w2p = jnp.pad(w2, ((0, 0), (0, 127)))
    b2p = jnp.broadcast_to(b2.reshape(1, 1), (1, 128))
    return pl.pallas_call(
        _tc_final_body,
        grid=grid,
        in_specs=[_row_spec(bm, 128), _row_spec(bm, 128), _row_spec(bm, 128),
                  _full_spec((128, 512)), _full_spec((128, 512)),
                  _full_spec((1, 512)),
                  _full_spec((1, 128)), _full_spec((1, 128)),
                  _full_spec((1, 128)), _full_spec((1, 128)),
                  _full_spec((1, 128)), _full_spec((1, 128)),
                  _row_spec(bm, 128),
                  _full_spec((128, 128)), _full_spec((1, 128)),
                  _full_spec((1, 128)),
                  _full_spec((128, 128)), _full_spec((1, 128))],
        out_specs=[_row_spec(bm, 128), _row_spec(bm, 128), _row_spec(bm, 128)],
        out_shape=out_shape,
        compiler_params=pltpu.CompilerParams(
            dimension_semantics=("arbitrary",)),
    )(ax, ah, c, wx, wh, b.reshape(1, 512),
      gh.reshape(1, 128), bh.reshape(1, 128),
      gc.reshape(1, 128), bc.reshape(1, 128),
      go.reshape(1, 128), bo.reshape(1, 128),
      skip128, w1[:128], w1[128:129], b1.reshape(1, 128),
      w2p, b2p)


def kernel(X, edge_index, edge_weight, skip, H, C, params):
    n, feat = X.shape
    e = edge_weight.shape[0]
    hid = H.shape[2]
    nlayers = H.shape[0]

    rpw = (-(-n // NS) + 7) // 8 * 8
    npad = rpw * NS
    cps = -(-e // (NS * CHUNK))
    epad = cps * NS * CHUNK
    pad = epad - e

    src = edge_index[0]
    dst = edge_index[1]
    zi = jnp.zeros((pad,), jnp.int32)
    src_p = jnp.concatenate([src, zi])
    dst_p = jnp.concatenate([dst, zi])
    ew_p = jnp.concatenate([edge_weight, jnp.zeros((pad,), jnp.float32)])
    src2 = jnp.concatenate([src_p, src_p + n])
    dst2 = jnp.concatenate([dst_p, dst_p])
    ew2 = jnp.concatenate([ew_p, ew_p])

    bm = 1000 if n % 1000 == 0 else 8
    agg = functools.partial(_sc_aggregate, cps=cps, rpw=rpw, npad=npad)

    x_in = X
    hn, cn = [], []
    for l in range(nlayers):
        table = jnp.concatenate([x_in, H[l]], axis=0)
        a = agg(table, src2, dst2, ew2)
        ax = a[:n]
        ah = a[npad:npad + n]
        if l < nlayers - 1:
            h_n, c_n = _tc_cell(
                ax, ah, C[l], params['Wx'][l], params['Wh'][l], params['b'][l],
                params['ln_h'][0], params['ln_h'][1],
                params['ln_c'][0], params['ln_c'][1], bm=bm)
        else:
            h_n, c_n, out128 = _tc_final(
                ax, ah, C[l], params['Wx'][l], params['Wh'][l], params['b'][l],
                params['ln_h'][0], params['ln_h'][1],
                params['ln_c'][0], params['ln_c'][1],
                params['ln_o'][0], params['ln_o'][1],
                skip, params['W1'], params['b1'], params['W2'], params['b2'],
                bm=bm)
            out = out128[:, :1]
        hn.append(h_n)
        cn.append(c_n)
        x_in = h_n

    return (out, jnp.stack(hn), jnp.stack(cn))
